# baseline (speedup 1.0000x reference)
import functools

import jax
import jax.numpy as jnp
from jax import lax
from jax.experimental import pallas as pl
from jax.experimental.pallas import tpu as pltpu
from jax.experimental.pallas import tpu_sc as plsc

E_DIM = 32
N_HEADS = 4
HEAD_DIM = E_DIM // N_HEADS
NUM_EXPERTS = 64
NOISE_EPS = 0.01

NC = 2
NS = 16
LANES = 16
NW = NC * NS
NEG = -3.0e38


def _prologue_body(tid_ref, emb_ref, ipw_ref, ipb_ref, ek_ref, fgw_ref,
                   fgb_ref, fnw_ref, fnb_ref, cs_ref):
    tid = tid_ref[...].reshape(1, 1)
    row_ids = jax.lax.broadcasted_iota(jnp.int32, (6, 1), 0)
    e_rows = jnp.where(row_ids == tid, emb_ref[...], 0.0)
    e = jnp.sum(e_rows, axis=0, keepdims=True)

    wq = ipw_ref[0:E_DIM, :]
    wk = ipw_ref[E_DIM:2 * E_DIM, :]
    bq = ipb_ref[0, 0:E_DIM]
    bk = ipb_ref[0, E_DIM:2 * E_DIM]

    dn = (((1,), (1,)), ((), ()))
    q = jax.lax.dot_general(e, wq, dn,
                            preferred_element_type=jnp.float32) + bq[None, :]
    k = jax.lax.dot_general(ek_ref[...], wk, dn,
                            preferred_element_type=jnp.float32) + bk[None, :]

    s_full = k * q
    d_ids = jax.lax.broadcasted_iota(jnp.int32, (E_DIM, N_HEADS), 0)
    h_ids = jax.lax.broadcasted_iota(jnp.int32, (E_DIM, N_HEADS), 1)
    head_mask = ((d_ids // HEAD_DIM) == h_ids).astype(jnp.float32)
    dn0 = (((1,), (0,)), ((), ()))
    scores = jax.lax.dot_general(s_full, head_mask, dn0,
                                 preferred_element_type=jnp.float32)
    scores = scores / jnp.sqrt(jnp.float32(HEAD_DIM))

    attn = jax.nn.softmax(scores, axis=0)
    avg = jnp.mean(attn, axis=1, keepdims=True)
    w = jax.nn.softmax(avg, axis=0).reshape(1, E_DIM)

    clean = jax.lax.dot_general(w, fgw_ref[...], dn,
                                preferred_element_type=jnp.float32)
    clean = clean + fgb_ref[...]
    raw = jax.lax.dot_general(w, fnw_ref[...], dn,
                              preferred_element_type=jnp.float32)
    raw = raw + fnb_ref[...]
    std = jnp.maximum(raw, 0.0) + jnp.log1p(jnp.exp(-jnp.abs(raw)))
    cs_ref[0:2, :] = jnp.zeros((2, NUM_EXPERTS), jnp.float32)
    cs_ref[2:3, :] = clean
    cs_ref[3:4, :] = std + NOISE_EPS


def _prologue(tid, taskID_embed, in_proj_weight, ipb, expert_keys,
              fc_gate_w, fgb, fc_noise_w, fnb):
    full = lambda shape: pl.BlockSpec(shape, lambda: (0,) * len(shape))
    return pl.pallas_call(
        _prologue_body,
        in_specs=[
            full((1, 1)), full((6, E_DIM)), full((3 * E_DIM, E_DIM)),
            full((1, 3 * E_DIM)), full((E_DIM, E_DIM)),
            full((NUM_EXPERTS, E_DIM)), full((1, NUM_EXPERTS)),
            full((NUM_EXPERTS, E_DIM)), full((1, NUM_EXPERTS)),
        ],
        out_specs=full((4, NUM_EXPERTS)),
        out_shape=jax.ShapeDtypeStruct((4, NUM_EXPERTS), jnp.float32),
    )(tid, taskID_embed, in_proj_weight, ipb, expert_keys,
      fc_gate_w, fgb, fc_noise_w, fnb)


def _sc_routing_body(rows_w, cs_hbm, noise_hbm, pk_hbm, g0_hbm, g1_hbm,
                     chunk, cs_v, csb, pko, g0o, g1o,
                     sem_in0, sem_in1):
    wid = lax.axis_index("s") * NC + lax.axis_index("c")
    fbase = wid * rows_w * NUM_EXPERTS
    fq = (rows_w // 4) * NUM_EXPERTS
    n_groups = rows_w // LANES
    qg = n_groups // 4
    lane = lax.iota(jnp.int32, LANES)

    pltpu.sync_copy(cs_hbm, cs_v)
    sems = (sem_in0, sem_in1)
    ins = [pltpu.async_copy(noise_hbm.at[pl.ds(fbase + q * 2 * fq, 2 * fq)],
                            chunk.at[pl.ds(q * 2 * fq, 2 * fq)], sems[q])
           for q in range(2)]

    two_i = jnp.full((LANES,), 2, jnp.int32)
    three_i = jnp.full((LANES,), 3, jnp.int32)
    for e in range(NUM_EXPERTS):
        col = jnp.full((LANES,), e, jnp.int32)
        csb[e] = plsc.load_gather(cs_v, [two_i, col])
        csb[NUM_EXPERTS + e] = plsc.load_gather(cs_v, [three_i, col])

    def _group(g, _):
        rows_fa = (g * 2 * LANES + lane) * NUM_EXPERTS
        rows_fb = rows_fa + LANES * NUM_EXPERTS
        m0a = jnp.full((LANES,), NEG, jnp.float32)
        m1a = jnp.full((LANES,), NEG, jnp.float32)
        i0a = jnp.zeros((LANES,), jnp.int32)
        i1a = jnp.zeros((LANES,), jnp.int32)
        m0b, m1b, i0b, i1b = m0a, m1a, i0a, i1a
        for e in range(NUM_EXPERTS):
            col = jnp.full((LANES,), e, jnp.int32)
            std_e = csb[NUM_EXPERTS + e]
            cln_e = csb[e]
            xa = plsc.load_gather(chunk, [rows_fa + col])
            xb = plsc.load_gather(chunk, [rows_fb + col])
            va = xa * std_e + cln_e
            vb = xb * std_e + cln_e
            ga0 = va > m0a
            ga1 = va > m1a
            m1a = jnp.where(ga0, m0a, jnp.where(ga1, va, m1a))
            i1a = jnp.where(ga0, i0a, jnp.where(ga1, col, i1a))
            m0a = jnp.where(ga0, va, m0a)
            i0a = jnp.where(ga0, col, i0a)
            gb0 = vb > m0b
            gb1 = vb > m1b
            m1b = jnp.where(gb0, m0b, jnp.where(gb1, vb, m1b))
            i1b = jnp.where(gb0, i0b, jnp.where(gb1, col, i1b))
            m0b = jnp.where(gb0, vb, m0b)
            i0b = jnp.where(gb0, col, i0b)
        for off, (m0, m1, i0, i1) in (
                (0, (m0a, m1a, i0a, i1a)), (LANES, (m0b, m1b, i0b, i1b))):
            e1 = jnp.exp(m1 - m0)
            g0 = 1.0 / (1.0 + e1)
            g1 = e1 / (1.0 + e1)
            pko[pl.ds(g * 2 * LANES + off, LANES)] = i0 | (i1 << 8)
            g0o[pl.ds(g * 2 * LANES + off, LANES)] = g0
            g1o[pl.ds(g * 2 * LANES + off, LANES)] = g1
        return _

    for q in range(2):
        ins[q].wait()
        lax.fori_loop(q * qg, (q + 1) * qg, _group, None)

    base = wid * rows_w
    pltpu.sync_copy(pko, pk_hbm.at[pl.ds(base, rows_w)])
    pltpu.sync_copy(g0o, g0_hbm.at[pl.ds(base, rows_w)])
    pltpu.sync_copy(g1o, g1_hbm.at[pl.ds(base, rows_w)])


def _sc_routing(cs, noise_flat, B):
    rows_w = B // NW
    mesh = plsc.VectorSubcoreMesh(core_axis_name="c", subcore_axis_name="s",
                                  num_cores=NC, num_subcores=NS)
    return pl.kernel(
        functools.partial(_sc_routing_body, rows_w),
        out_type=[
            jax.ShapeDtypeStruct((B,), jnp.int32),
            jax.ShapeDtypeStruct((B,), jnp.float32),
            jax.ShapeDtypeStruct((B,), jnp.float32),
        ],
        mesh=mesh,
        scratch_types=[
            pltpu.VMEM((rows_w * NUM_EXPERTS,), jnp.float32),
            pltpu.VMEM((4, NUM_EXPERTS), jnp.float32),
            pltpu.VMEM((2 * NUM_EXPERTS, LANES), jnp.float32),
            pltpu.VMEM((rows_w,), jnp.int32),
            pltpu.VMEM((rows_w,), jnp.float32),
            pltpu.VMEM((rows_w,), jnp.float32),
            pltpu.SemaphoreType.DMA,
            pltpu.SemaphoreType.DMA,
        ],
        compiler_params=pltpu.CompilerParams(needs_layout_passes=False),
    )(cs, noise_flat)


_EBLK = 1024


def _expand_body(pk_ref, g0_ref, g1_ref, gt_ref, load_ref):
    step = pl.program_id(0)
    pk = pk_ref[...].reshape(1, _EBLK)
    i0 = pk & 0xFF
    i1 = pk >> 8
    g0 = g0_ref[...].reshape(1, _EBLK)
    g1 = g1_ref[...].reshape(1, _EBLK)
    erow = jax.lax.broadcasted_iota(jnp.int32, (NUM_EXPERTS, _EBLK), 0)
    gates_t = jnp.where(erow == i0, g0, 0.0) + jnp.where(erow == i1, g1, 0.0)
    gt_ref[...] = gates_t

    @pl.when(step == 0)
    def _init():
        load_ref[...] = jnp.zeros_like(load_ref)

    load_ref[...] += jnp.sum(gates_t, axis=1, keepdims=True)


def _expand(pk, g0, g1, B):
    n_blk = B // _EBLK
    return pl.pallas_call(
        _expand_body,
        grid=(n_blk,),
        in_specs=[
            pl.BlockSpec((_EBLK,), lambda i: (i,)),
            pl.BlockSpec((_EBLK,), lambda i: (i,)),
            pl.BlockSpec((_EBLK,), lambda i: (i,)),
        ],
        out_specs=[
            pl.BlockSpec((NUM_EXPERTS, _EBLK), lambda i: (0, i)),
            pl.BlockSpec((NUM_EXPERTS, 1), lambda i: (0, 0)),
        ],
        out_shape=[
            jax.ShapeDtypeStruct((NUM_EXPERTS, B), jnp.float32),
            jax.ShapeDtypeStruct((NUM_EXPERTS, 1), jnp.float32),
        ],
        compiler_params=pltpu.CompilerParams(
            dimension_semantics=("arbitrary",)),
    )(pk, g0, g1)


def kernel(task_id, bsz, taskID_embed, in_proj_weight, in_proj_bias,
           out_proj_weight, out_proj_bias, expert_keys,
           fc_gate_w, fc_gate_b, fc_noise_w, fc_noise_b, noise):
    del bsz, out_proj_weight, out_proj_bias
    tid = jnp.asarray(task_id, jnp.int32).reshape(1, 1)
    ipb = in_proj_bias.reshape(1, -1)
    fgb = fc_gate_b.reshape(1, -1)
    fnb = fc_noise_b.reshape(1, -1)

    B = noise.shape[0]
    cs = _prologue(tid, taskID_embed, in_proj_weight, ipb, expert_keys,
                   fc_gate_w, fgb, fc_noise_w, fnb)
    pk, g0, g1 = _sc_routing(cs, noise.reshape(-1), B)
    gates_t, load = _expand(pk, g0, g1, B)
    return gates_t.T, load.reshape(NUM_EXPERTS)


# speedup vs baseline: 1.0408x; 1.0408x over previous
import functools

import jax
import jax.numpy as jnp
from jax import lax
from jax.experimental import pallas as pl
from jax.experimental.pallas import tpu as pltpu
from jax.experimental.pallas import tpu_sc as plsc

E_DIM = 32
N_HEADS = 4
HEAD_DIM = E_DIM // N_HEADS
NUM_EXPERTS = 64
NOISE_EPS = 0.01

NC = 2
NS = 16
LANES = 16
NW = NC * NS
NEG = -3.0e38


def _prologue_body(tid_ref, emb_ref, ipw_ref, ipb_ref, ek_ref, fgw_ref,
                   fgb_ref, fnw_ref, fnb_ref, cs_ref):
    tid = tid_ref[...].reshape(1, 1)
    row_ids = jax.lax.broadcasted_iota(jnp.int32, (6, 1), 0)
    e_rows = jnp.where(row_ids == tid, emb_ref[...], 0.0)
    e = jnp.sum(e_rows, axis=0, keepdims=True)

    wq = ipw_ref[0:E_DIM, :]
    wk = ipw_ref[E_DIM:2 * E_DIM, :]
    bq = ipb_ref[0, 0:E_DIM]
    bk = ipb_ref[0, E_DIM:2 * E_DIM]

    dn = (((1,), (1,)), ((), ()))
    q = jax.lax.dot_general(e, wq, dn,
                            preferred_element_type=jnp.float32) + bq[None, :]
    k = jax.lax.dot_general(ek_ref[...], wk, dn,
                            preferred_element_type=jnp.float32) + bk[None, :]

    s_full = k * q
    d_ids = jax.lax.broadcasted_iota(jnp.int32, (E_DIM, N_HEADS), 0)
    h_ids = jax.lax.broadcasted_iota(jnp.int32, (E_DIM, N_HEADS), 1)
    head_mask = ((d_ids // HEAD_DIM) == h_ids).astype(jnp.float32)
    dn0 = (((1,), (0,)), ((), ()))
    scores = jax.lax.dot_general(s_full, head_mask, dn0,
                                 preferred_element_type=jnp.float32)
    scores = scores / jnp.sqrt(jnp.float32(HEAD_DIM))

    attn = jax.nn.softmax(scores, axis=0)
    avg = jnp.mean(attn, axis=1, keepdims=True)
    w = jax.nn.softmax(avg, axis=0).reshape(1, E_DIM)

    clean = jax.lax.dot_general(w, fgw_ref[...], dn,
                                preferred_element_type=jnp.float32)
    clean = clean + fgb_ref[...]
    raw = jax.lax.dot_general(w, fnw_ref[...], dn,
                              preferred_element_type=jnp.float32)
    raw = raw + fnb_ref[...]
    std = jnp.maximum(raw, 0.0) + jnp.log1p(jnp.exp(-jnp.abs(raw)))
    cs_ref[0:2, :] = jnp.zeros((2, NUM_EXPERTS), jnp.float32)
    cs_ref[2:3, :] = clean
    cs_ref[3:4, :] = std + NOISE_EPS


def _prologue(tid, taskID_embed, in_proj_weight, ipb, expert_keys,
              fc_gate_w, fgb, fc_noise_w, fnb):
    full = lambda shape: pl.BlockSpec(shape, lambda: (0,) * len(shape))
    return pl.pallas_call(
        _prologue_body,
        in_specs=[
            full((1, 1)), full((6, E_DIM)), full((3 * E_DIM, E_DIM)),
            full((1, 3 * E_DIM)), full((E_DIM, E_DIM)),
            full((NUM_EXPERTS, E_DIM)), full((1, NUM_EXPERTS)),
            full((NUM_EXPERTS, E_DIM)), full((1, NUM_EXPERTS)),
        ],
        out_specs=full((4, NUM_EXPERTS)),
        out_shape=jax.ShapeDtypeStruct((4, NUM_EXPERTS), jnp.float32),
    )(tid, taskID_embed, in_proj_weight, ipb, expert_keys,
      fc_gate_w, fgb, fc_noise_w, fnb)


def _sc_routing_body(rows_w, cs_hbm, noise_hbm, pk_hbm, g0_hbm, g1_hbm,
                     chunk, cs_v, csb, pko, g0o, g1o,
                     sem_in0, sem_in1):
    wid = lax.axis_index("s") * NC + lax.axis_index("c")
    fbase = wid * rows_w * NUM_EXPERTS
    fq = (rows_w // 4) * NUM_EXPERTS
    n_groups = rows_w // LANES
    qg = n_groups // 4
    lane = lax.iota(jnp.int32, LANES)

    pltpu.sync_copy(cs_hbm, cs_v)
    sems = (sem_in0, sem_in1)
    ins = [pltpu.async_copy(noise_hbm.at[pl.ds(fbase + q * 2 * fq, 2 * fq)],
                            chunk.at[pl.ds(q * 2 * fq, 2 * fq)], sems[q])
           for q in range(2)]

    two_i = jnp.full((LANES,), 2, jnp.int32)
    three_i = jnp.full((LANES,), 3, jnp.int32)
    for e in range(NUM_EXPERTS):
        col = jnp.full((LANES,), e, jnp.int32)
        csb[e] = plsc.load_gather(cs_v, [two_i, col])
        csb[NUM_EXPERTS + e] = plsc.load_gather(cs_v, [three_i, col])

    def _group(g, _):
        rows_f = (g * LANES + lane) * NUM_EXPERTS
        m0 = jnp.full((LANES,), NEG, jnp.float32)
        m1 = jnp.full((LANES,), NEG, jnp.float32)
        i0 = jnp.zeros((LANES,), jnp.int32)
        i1 = jnp.zeros((LANES,), jnp.int32)
        for e in range(NUM_EXPERTS):
            col = jnp.full((LANES,), e, jnp.int32)
            x = plsc.load_gather(chunk, [rows_f + col])
            v = x * csb[NUM_EXPERTS + e] + csb[e]
            gt0 = v > m0
            gt1 = v > m1
            m1 = jnp.where(gt0, m0, jnp.where(gt1, v, m1))
            i1 = jnp.where(gt0, i0, jnp.where(gt1, col, i1))
            m0 = jnp.where(gt0, v, m0)
            i0 = jnp.where(gt0, col, i0)
        e1 = jnp.exp(m1 - m0)
        g0 = 1.0 / (1.0 + e1)
        g1 = e1 / (1.0 + e1)
        pko[pl.ds(g * LANES, LANES)] = i0 | (i1 << 8)
        g0o[pl.ds(g * LANES, LANES)] = g0
        g1o[pl.ds(g * LANES, LANES)] = g1
        return _

    for q in range(2):
        ins[q].wait()
        lax.fori_loop(q * 2 * qg, (q + 1) * 2 * qg, _group, None)

    base = wid * rows_w
    pltpu.sync_copy(pko, pk_hbm.at[pl.ds(base, rows_w)])
    pltpu.sync_copy(g0o, g0_hbm.at[pl.ds(base, rows_w)])
    pltpu.sync_copy(g1o, g1_hbm.at[pl.ds(base, rows_w)])


def _sc_routing(cs, noise_flat, B):
    rows_w = B // NW
    mesh = plsc.VectorSubcoreMesh(core_axis_name="c", subcore_axis_name="s",
                                  num_cores=NC, num_subcores=NS)
    return pl.kernel(
        functools.partial(_sc_routing_body, rows_w),
        out_type=[
            jax.ShapeDtypeStruct((B,), jnp.int32),
            jax.ShapeDtypeStruct((B,), jnp.float32),
            jax.ShapeDtypeStruct((B,), jnp.float32),
        ],
        mesh=mesh,
        scratch_types=[
            pltpu.VMEM((rows_w * NUM_EXPERTS,), jnp.float32),
            pltpu.VMEM((4, NUM_EXPERTS), jnp.float32),
            pltpu.VMEM((2 * NUM_EXPERTS, LANES), jnp.float32),
            pltpu.VMEM((rows_w,), jnp.int32),
            pltpu.VMEM((rows_w,), jnp.float32),
            pltpu.VMEM((rows_w,), jnp.float32),
            pltpu.SemaphoreType.DMA,
            pltpu.SemaphoreType.DMA,
        ],
        compiler_params=pltpu.CompilerParams(needs_layout_passes=False),
    )(cs, noise_flat)


_EBLK = 1024


def _expand_body(pk_ref, g0_ref, g1_ref, gt_ref, load_ref):
    step = pl.program_id(0)
    pk = pk_ref[...].reshape(1, _EBLK)
    i0 = pk & 0xFF
    i1 = pk >> 8
    g0 = g0_ref[...].reshape(1, _EBLK)
    g1 = g1_ref[...].reshape(1, _EBLK)
    erow = jax.lax.broadcasted_iota(jnp.int32, (NUM_EXPERTS, _EBLK), 0)
    gates_t = jnp.where(erow == i0, g0, 0.0) + jnp.where(erow == i1, g1, 0.0)
    gt_ref[...] = gates_t

    @pl.when(step == 0)
    def _init():
        load_ref[...] = jnp.zeros_like(load_ref)

    load_ref[...] += jnp.sum(gates_t, axis=1, keepdims=True)


def _expand(pk, g0, g1, B):
    n_blk = B // _EBLK
    return pl.pallas_call(
        _expand_body,
        grid=(n_blk,),
        in_specs=[
            pl.BlockSpec((_EBLK,), lambda i: (i,)),
            pl.BlockSpec((_EBLK,), lambda i: (i,)),
            pl.BlockSpec((_EBLK,), lambda i: (i,)),
        ],
        out_specs=[
            pl.BlockSpec((NUM_EXPERTS, _EBLK), lambda i: (0, i)),
            pl.BlockSpec((NUM_EXPERTS, 1), lambda i: (0, 0)),
        ],
        out_shape=[
            jax.ShapeDtypeStruct((NUM_EXPERTS, B), jnp.float32),
            jax.ShapeDtypeStruct((NUM_EXPERTS, 1), jnp.float32),
        ],
        compiler_params=pltpu.CompilerParams(
            dimension_semantics=("arbitrary",)),
    )(pk, g0, g1)


def kernel(task_id, bsz, taskID_embed, in_proj_weight, in_proj_bias,
           out_proj_weight, out_proj_bias, expert_keys,
           fc_gate_w, fc_gate_b, fc_noise_w, fc_noise_b, noise):
    del bsz, out_proj_weight, out_proj_bias
    tid = jnp.asarray(task_id, jnp.int32).reshape(1, 1)
    ipb = in_proj_bias.reshape(1, -1)
    fgb = fc_gate_b.reshape(1, -1)
    fnb = fc_noise_b.reshape(1, -1)

    B = noise.shape[0]
    cs = _prologue(tid, taskID_embed, in_proj_weight, ipb, expert_keys,
                   fc_gate_w, fgb, fc_noise_w, fnb)
    pk, g0, g1 = _sc_routing(cs, noise.reshape(-1), B)
    gates_t, load = _expand(pk, g0, g1, B)
    return gates_t.T, load.reshape(NUM_EXPERTS)
